# Initial kernel scaffold; baseline (speedup 1.0000x reference)
import jax
import jax.numpy as jnp
from jax.experimental import pallas as pl


def kernel(x, edge_index, Wl_first, Wr_first, b_first, Wl_mid, Wr_mid, b_mid, Wl_last, Wr_last, b_last):
    raise NotImplementedError("write your pallas kernel here")


# speedup vs baseline: 3.6655x; 3.6655x over previous
import functools

import jax
import jax.numpy as jnp
from jax import lax
from jax._src import config as _jax_config
from jax.experimental import pallas as pl
from jax.experimental.pallas import tpu as pltpu
from jax.experimental.pallas import tpu_sc as plsc

N = 50000
E = 1600000
HID = 256
NCH = 16
CW = 16
NT = 16
NPAD = 50048
RPT = NPAD // NT
K = 128
NB = 8
SB = K * NB
EPT = 100352
NSB = EPT // SB
EPAD = EPT * NT
EPT2 = EPAD // (2 * NT)
NSB2 = EPT2 // SB
BN = 400
NBLK = N // BN


def _mesh():
    return plsc.VectorSubcoreMesh(core_axis_name="c", subcore_axis_name="s",
                                  num_cores=2, num_subcores=NT)


def _edge_loop(table, src_ref, dst_ref, e_base, nsb,
               srcj, dstj, rowsj, acc, semi, semg, sema):
    def body(b, carry):
        e0 = e_base + b * jnp.int32(SB)
        ils = [pltpu.async_copy(src_ref.at[pl.ds(e0 + jnp.int32(j * K), K)],
                                srcj[j], semi) for j in range(NB)]
        ild = [pltpu.async_copy(dst_ref.at[pl.ds(e0 + jnp.int32(j * K), K)],
                                dstj[j], semi) for j in range(NB)]
        gs = []
        for j in range(NB):
            ils[j].wait()
            gs.append(pltpu.async_copy(table.at[srcj[j]], rowsj[j], semg))
        adds = []
        for j in range(NB):
            ild[j].wait()
            gs[j].wait()
            adds.append(pltpu.async_copy(rowsj[j], acc.at[dstj[j]],
                                         sema, add=True))
        for d in adds:
            d.wait()
        return carry

    lax.fori_loop(jnp.int32(0), jnp.int32(nsb), body, jnp.int32(0))


def _sc_scratch():
    return ([pltpu.VMEM((K,), jnp.int32) for _ in range(NB)]
            + [pltpu.VMEM((K,), jnp.int32) for _ in range(NB)]
            + [pltpu.VMEM((K, CW), jnp.float32) for _ in range(NB)]
            + [pltpu.VMEM_SHARED((NPAD, CW), jnp.float32),
               pltpu.SemaphoreType.DMA,
               pltpu.SemaphoreType.DMA,
               pltpu.SemaphoreType.DMA])


def _make_sc_agg():
    out_type = tuple(jax.ShapeDtypeStruct((NPAD, CW), jnp.float32)
                     for _ in range(NCH))

    @functools.partial(pl.kernel, out_type=out_type, mesh=_mesh(),
                       scratch_types=_sc_scratch(),
                       compiler_params=pltpu.CompilerParams(
                           use_tc_tiling_on_sc=False))
    def agg(src_ref, dst_ref, zeros_ref, *rest):
        tables = list(rest[:NCH])
        outs = list(rest[NCH:2 * NCH])
        sc = rest[2 * NCH:]
        srcj = list(sc[:NB])
        dstj = list(sc[NB:2 * NB])
        rowsj = list(sc[2 * NB:3 * NB])
        acc, semi, semg, sema = sc[3 * NB:]
        core = lax.axis_index("c")
        sub = lax.axis_index("s")
        e_base = sub * jnp.int32(EPT)
        for cc in range(2):
            @pl.when(core == cc)
            def _(cc=cc):
                for ci in range(NCH // 2):
                    c = cc * (NCH // 2) + ci
                    pltpu.sync_copy(zeros_ref,
                                    acc.at[pl.ds(sub * jnp.int32(RPT), RPT)])
                    plsc.subcore_barrier()
                    _edge_loop(tables[c], src_ref, dst_ref, e_base, NSB,
                               srcj, dstj, rowsj, acc, semi, semg, sema)
                    plsc.subcore_barrier()
                    pltpu.sync_copy(acc.at[pl.ds(sub * jnp.int32(RPT), RPT)],
                                    outs[c].at[pl.ds(sub * jnp.int32(RPT), RPT)])
                    plsc.subcore_barrier()

    return agg


def _make_sc_first():
    out_type = (jax.ShapeDtypeStruct((NPAD, CW), jnp.float32),
                jax.ShapeDtypeStruct((NPAD, CW), jnp.float32))

    @functools.partial(pl.kernel, out_type=out_type, mesh=_mesh(),
                       scratch_types=_sc_scratch(),
                       compiler_params=pltpu.CompilerParams(
                           use_tc_tiling_on_sc=False))
    def first(src_ref, dst_ref, zeros_ref, xp_ref, o0_ref, o1_ref, *sc):
        srcj = list(sc[:NB])
        dstj = list(sc[NB:2 * NB])
        rowsj = list(sc[2 * NB:3 * NB])
        acc, semi, semg, sema = sc[3 * NB:]
        core = lax.axis_index("c")
        sub = lax.axis_index("s")
        outs = [o0_ref, o1_ref]
        for cc in range(2):
            @pl.when(core == cc)
            def _(cc=cc):
                e_base = (jnp.int32(cc * NT) + sub) * jnp.int32(EPT2)
                pltpu.sync_copy(zeros_ref,
                                acc.at[pl.ds(sub * jnp.int32(RPT), RPT)])
                plsc.subcore_barrier()
                _edge_loop(xp_ref, src_ref, dst_ref, e_base, NSB2,
                           srcj, dstj, rowsj, acc, semi, semg, sema)
                plsc.subcore_barrier()
                pltpu.sync_copy(acc.at[pl.ds(sub * jnp.int32(RPT), RPT)],
                                outs[cc].at[pl.ds(sub * jnp.int32(RPT), RPT)])

    return first


def _chunk_specs():
    return [pl.BlockSpec((BN, CW), lambda i: (i, 0)) for _ in range(NCH)]


def _chunk_shapes():
    return tuple(jax.ShapeDtypeStruct((NPAD, CW), jnp.float32)
                 for _ in range(NCH))


def _tc_dense_mid(*refs):
    s = refs[0:NCH]
    h = refs[NCH:2 * NCH]
    deg_ref, wlt_ref, wrt_ref, b_ref = refs[2 * NCH:2 * NCH + 4]
    outs = refs[2 * NCH + 4:]
    scat = jnp.concatenate([s[c][...] for c in range(NCH)], axis=1)
    hcat = jnp.concatenate([h[c][...] for c in range(NCH)], axis=1)
    inv = 1.0 / jnp.maximum(deg_ref[...], 1.0)
    agg = scat * inv
    o = (jnp.dot(agg, wlt_ref[...], preferred_element_type=jnp.float32)
         + jnp.dot(hcat, wrt_ref[...], preferred_element_type=jnp.float32)
         + b_ref[...])
    o = jnp.maximum(o, 0.0)
    for c in range(NCH):
        outs[c][...] = o[:, c * CW:(c + 1) * CW]


def _make_tc_mid():
    return pl.pallas_call(
        _tc_dense_mid,
        grid=(NBLK,),
        in_specs=(_chunk_specs() + _chunk_specs() + [
            pl.BlockSpec((BN, 1), lambda i: (i, 0)),
            pl.BlockSpec((HID, HID), lambda i: (0, 0)),
            pl.BlockSpec((HID, HID), lambda i: (0, 0)),
            pl.BlockSpec((1, HID), lambda i: (0, 0)),
        ]),
        out_specs=_chunk_specs(),
        out_shape=_chunk_shapes(),
    )


def _tc_dense_first(a0_ref, a1_ref, x_ref, wlt_ref, wrt_ref, b_ref, *outs):
    s = a0_ref[...] + a1_ref[...]
    deg = s[:, 2:3]
    inv = 1.0 / jnp.maximum(deg, 1.0)
    agg = s[:, 0:2] * inv
    o = (jnp.dot(agg, wlt_ref[...], preferred_element_type=jnp.float32)
         + jnp.dot(x_ref[...], wrt_ref[...], preferred_element_type=jnp.float32)
         + b_ref[...])
    o = jnp.maximum(o, 0.0)
    for c in range(NCH):
        outs[c][...] = o[:, c * CW:(c + 1) * CW]
    outs[NCH][...] = deg


def _make_tc_first():
    return pl.pallas_call(
        _tc_dense_first,
        grid=(NBLK,),
        in_specs=[
            pl.BlockSpec((BN, CW), lambda i: (i, 0)),
            pl.BlockSpec((BN, CW), lambda i: (i, 0)),
            pl.BlockSpec((BN, 2), lambda i: (i, 0)),
            pl.BlockSpec((2, HID), lambda i: (0, 0)),
            pl.BlockSpec((2, HID), lambda i: (0, 0)),
            pl.BlockSpec((1, HID), lambda i: (0, 0)),
        ],
        out_specs=_chunk_specs() + [pl.BlockSpec((BN, 1), lambda i: (i, 0))],
        out_shape=_chunk_shapes() + (jax.ShapeDtypeStruct((N, 1), jnp.float32),),
    )


def _tc_dense_last(*refs):
    s = refs[0:NCH]
    h = refs[NCH:2 * NCH]
    deg_ref, wlt_ref, wrt_ref, b_ref, out_ref = refs[2 * NCH:]
    scat = jnp.concatenate([s[c][...] for c in range(NCH)], axis=1)
    hcat = jnp.concatenate([h[c][...] for c in range(NCH)], axis=1)
    inv = 1.0 / jnp.maximum(deg_ref[...], 1.0)
    agg = scat * inv
    o = (jnp.dot(agg, wlt_ref[...], preferred_element_type=jnp.float32)
         + jnp.dot(hcat, wrt_ref[...], preferred_element_type=jnp.float32)
         + b_ref[...])
    out_ref[...] = o


def _make_tc_last():
    return pl.pallas_call(
        _tc_dense_last,
        grid=(NBLK,),
        in_specs=(_chunk_specs() + _chunk_specs() + [
            pl.BlockSpec((BN, 1), lambda i: (i, 0)),
            pl.BlockSpec((HID, 1), lambda i: (0, 0)),
            pl.BlockSpec((HID, 1), lambda i: (0, 0)),
            pl.BlockSpec((1, 1), lambda i: (0, 0)),
        ]),
        out_specs=pl.BlockSpec((BN, 1), lambda i: (i, 0)),
        out_shape=jax.ShapeDtypeStruct((N, 1), jnp.float32),
    )


_sc_agg = _make_sc_agg()
_sc_first = _make_sc_first()
_tc_mid = _make_tc_mid()
_tc_first = _make_tc_first()
_tc_last = _make_tc_last()


def kernel(x, edge_index, Wl_first, Wr_first, b_first, Wl_mid, Wr_mid, b_mid,
           Wl_last, Wr_last, b_last):
    with _jax_config.enable_x64(False):
        return _kernel_x32(x, edge_index, Wl_first, Wr_first, b_first,
                           Wl_mid, Wr_mid, b_mid, Wl_last, Wr_last, b_last)


def _kernel_x32(x, edge_index, Wl_first, Wr_first, b_first, Wl_mid, Wr_mid,
                b_mid, Wl_last, Wr_last, b_last):
    ei = edge_index.astype(jnp.int32)
    src = jnp.concatenate([ei[0], jnp.zeros((EPAD - E,), jnp.int32)])
    dst = jnp.concatenate([ei[1], jnp.full((EPAD - E,), N, jnp.int32)])

    zc = jnp.zeros((RPT, CW), jnp.float32)

    xf = x.astype(jnp.float32)
    xp = jnp.zeros((NPAD, CW), jnp.float32)
    xp = xp.at[:N, 0:2].set(xf)
    xp = xp.at[:N, 2].set(1.0)

    wlt1 = Wl_first.T.astype(jnp.float32)
    wrt1 = Wr_first.T.astype(jnp.float32)
    b1 = b_first.reshape(1, HID).astype(jnp.float32)
    wltm = jnp.transpose(Wl_mid, (0, 2, 1)).astype(jnp.float32)
    wrtm = jnp.transpose(Wr_mid, (0, 2, 1)).astype(jnp.float32)
    bm = b_mid.reshape(b_mid.shape[0], 1, HID).astype(jnp.float32)
    wltL = Wl_last.T.astype(jnp.float32)
    wrtL = Wr_last.T.astype(jnp.float32)
    bL = b_last.reshape(1, 1).astype(jnp.float32)

    a0, a1 = _sc_first(src, dst, zc, xp)
    *h, deg = _tc_first(a0, a1, xf, wlt1, wrt1, b1)

    n_mid = Wl_mid.shape[0]
    for i in range(n_mid):
        s = _sc_agg(src, dst, zc, *h)
        h = _tc_mid(*s, *h, deg, wltm[i], wrtm[i], bm[i])

    s = _sc_agg(src, dst, zc, *h)
    out = _tc_last(*s, *h, deg, wltL, wrtL, bL)
    return out
